# Initial kernel scaffold; baseline (speedup 1.0000x reference)
import jax
import jax.numpy as jnp
from jax.experimental import pallas as pl


def kernel(x, edge_index, edge_attr, Wc, bc, W1, b1, W2, b2, W3, b3):
    raise NotImplementedError("write your pallas kernel here")


# speedup vs baseline: 14.7424x; 14.7424x over previous
import functools

import jax
import jax.numpy as jnp
from jax import lax
from jax.experimental import pallas as pl
from jax.experimental.pallas import tpu as pltpu
from jax.experimental.pallas import tpu_sc as plsc

N = 10000
E = 320000
D = 128
NC = 2
NS = 16
NW = NC * NS
EPW = E // NW
NB = EPW // 16
SB = 5
NSB = NB // SB
DEGR = 640
HALF = DEGR * 16 // 2
RPT = N // NS
ZR = 25
CHK = 2000
NCHK = EPW // CHK
SBC = CHK // (SB * 16)

_mesh = plsc.VectorSubcoreMesh(core_axis_name="c", subcore_axis_name="s")


def _rsqrt16(t):
    bits = lax.bitcast_convert_type(t, jnp.int32)
    y = lax.bitcast_convert_type(jnp.int32(0x5F3759DF) - (bits >> 1),
                                 jnp.float32)
    for _ in range(3):
        y = y * (1.5 - 0.5 * t * y * y)
    return y


@functools.partial(
    pl.kernel,
    out_type=jax.ShapeDtypeStruct((NC, DEGR, 16), jnp.float32),
    mesh=_mesh,
    compiler_params=pltpu.CompilerParams(needs_layout_passes=False, use_tc_tiling_on_sc=False),
    scratch_types=[
        pltpu.VMEM((EPW,), jnp.int32),
        pltpu.VMEM((EPW,), jnp.float32),
        pltpu.VMEM((16 * HALF + 16,), jnp.float32),
        pltpu.VMEM((DEGR, 16), jnp.float32),
        pltpu.VMEM((40, 16), jnp.float32),
        pltpu.VMEM((40, 16), jnp.float32),
        pltpu.VMEM_SHARED((NS, DEGR, 16), jnp.float32),
    ],
)
def _deg_kernel(dst_hbm, w_hbm, deg_out, dst_v, w_v, hist, deg_local,
                acc40, buf40, deg_sh):
    cid = lax.axis_index("c")
    sid = lax.axis_index("s")
    wid = sid * NC + cid

    pltpu.sync_copy(dst_hbm.at[wid], dst_v)
    pltpu.sync_copy(w_hbm.at[wid], w_v)

    zero = jnp.zeros((16,), jnp.float32)
    lane = lax.iota(jnp.int32, 16)

    def _zero_hist(i, _):
        hist[pl.ds(i * 16, 16)] = zero
        return 0
    lax.fori_loop(0, (16 * HALF + 16) // 16, _zero_hist, 0)

    for p in range(2):
        def _hist_blk(i, _):
            dvec = dst_v[pl.ds(i * 16, 16)]
            wvec = w_v[pl.ds(i * 16, 16)]
            rel = dvec - p * HALF
            mask = (rel >= 0) & (rel < HALF)
            idx = jnp.where(mask, lane * HALF + rel, 16 * HALF + lane)
            plsc.addupdate_scatter(hist, [idx], wvec)
            return 0
        lax.fori_loop(0, NB, _hist_blk, 0)

        def _red_row(r, _):
            acc = zero
            for j in range(16):
                acc = acc + hist[pl.ds(j * HALF + r * 16, 16)]
            deg_local[p * (HALF // 16) + r, :] = acc
            return 0
        lax.fori_loop(0, HALF // 16, _red_row, 0)
        if p == 0:
            lax.fori_loop(0, (16 * HALF + 16) // 16, _zero_hist, 0)

    pltpu.sync_copy(deg_local, deg_sh.at[sid])
    plsc.subcore_barrier()

    def _zero40(i, _):
        acc40[i, :] = zero
        return 0
    lax.fori_loop(0, 40, _zero40, 0)
    for t in range(NS):
        pltpu.sync_copy(deg_sh.at[t, pl.ds(sid * 40, 40)], buf40)

        def _add40(i, _):
            acc40[i, :] = acc40[i, :] + buf40[i, :]
            return 0
        lax.fori_loop(0, 40, _add40, 0)
    pltpu.sync_copy(acc40, deg_out.at[cid, pl.ds(sid * 40, 40)])


@functools.partial(
    pl.kernel,
    out_type=jax.ShapeDtypeStruct((NC, N, D), jnp.float32),
    mesh=_mesh,
    compiler_params=pltpu.CompilerParams(needs_layout_passes=False, use_tc_tiling_on_sc=False),
    scratch_types=[
        pltpu.VMEM((CHK,), jnp.int32),
        pltpu.VMEM((CHK,), jnp.int32),
        pltpu.VMEM((CHK,), jnp.float32),
        pltpu.VMEM((40, 16), jnp.float32),
        pltpu.VMEM((DEGR, 16), jnp.float32),
        pltpu.VMEM((SB * 16,), jnp.int32),
        pltpu.VMEM((SB * 16,), jnp.int32),
        pltpu.VMEM((SB * 16,), jnp.float32),
        pltpu.VMEM((SB * 16, D), jnp.float32),
        pltpu.VMEM((ZR, D), jnp.float32),
        pltpu.VMEM_SHARED((N, D), jnp.float32),
        pltpu.SemaphoreType.DMA,
    ],
)
def _msg_kernel(src_hbm, dst_hbm, w_hbm, h_hbm, deg_hbm, acc_out,
                src_c, dst_c, w_c, degbuf, dis_v, sidx, didx, fbuf, rows,
                zbuf, acc_sh, sem):
    cid = lax.axis_index("c")
    sid = lax.axis_index("s")
    wid = sid * NC + cid

    zero = jnp.zeros((16,), jnp.float32)

    pltpu.sync_copy(deg_hbm.at[0], dis_v)

    def _dis_chunk(b, _):
        pltpu.sync_copy(deg_hbm.at[1, pl.ds(b * 40, 40)], degbuf)

        def _dis_row(i, _):
            t = dis_v[b * 40 + i, :] + degbuf[i, :] + 1.0
            dis_v[b * 40 + i, :] = _rsqrt16(t)
            return 0
        lax.fori_loop(0, 40, _dis_row, 0)
        return 0
    lax.fori_loop(0, DEGR // 40, _dis_chunk, 0)

    def _zero_zbuf(i, _):
        for f in range(D // 16):
            zbuf[i, pl.ds(f * 16, 16)] = zero
        return 0
    lax.fori_loop(0, ZR, _zero_zbuf, 0)

    def _zero_acc(k, _):
        pltpu.sync_copy(zbuf, acc_sh.at[pl.ds(sid * RPT + k * ZR, ZR)])
        return 0
    lax.fori_loop(0, RPT // ZR, _zero_acc, 0)
    plsc.subcore_barrier()

    def _chunk(c, _):
        pltpu.sync_copy(src_hbm.at[wid, pl.ds(c * CHK, CHK)], src_c)
        pltpu.sync_copy(dst_hbm.at[wid, pl.ds(c * CHK, CHK)], dst_c)
        pltpu.sync_copy(w_hbm.at[wid, pl.ds(c * CHK, CHK)], w_c)

        def _super(s, _):
            def _factor(k, _):
                base = (s * SB + k) * 16
                sv = src_c[pl.ds(base, 16)]
                dv = dst_c[pl.ds(base, 16)]
                sidx[pl.ds(k * 16, 16)] = sv
                didx[pl.ds(k * 16, 16)] = dv
                f = w_c[pl.ds(base, 16)] * plsc.load_gather(
                    dis_v, [sv >> 4, sv & 15])
                fbuf[pl.ds(k * 16, 16)] = f
                return 0
            lax.fori_loop(0, SB, _factor, 0)

            pltpu.async_copy(h_hbm.at[sidx], rows, sem).wait()

            def _scale(j, _):
                fj = plsc.load_gather(fbuf, [jnp.full((16,), j, jnp.int32)])
                for f in range(D // 16):
                    rows[j, pl.ds(f * 16, 16)] = (
                        rows[j, pl.ds(f * 16, 16)] * fj)
                return 0
            lax.fori_loop(0, SB * 16, _scale, 0)

            pltpu.sync_copy(rows, acc_sh.at[didx], add=True)
            return 0
        lax.fori_loop(0, SBC, _super, 0)
        return 0
    lax.fori_loop(0, NCHK, _chunk, 0)
    plsc.subcore_barrier()

    def _out(k, _):
        pltpu.sync_copy(acc_sh.at[pl.ds(sid * RPT + k * ZR, ZR)], zbuf)
        pltpu.sync_copy(zbuf, acc_out.at[cid, pl.ds(sid * RPT + k * ZR, ZR)])
        return 0
    lax.fori_loop(0, RPT // ZR, _out, 0)


_RB = 2000


def _mm_body(x_ref, w_ref, o_ref):
    o_ref[...] = jnp.dot(x_ref[...], w_ref[...],
                         preferred_element_type=jnp.float32)


def _leaky(v):
    return jnp.where(v >= 0, v, 0.01 * v)


def _ep_body(acc_ref, h_ref, deg_ref, x_ref, bc_ref, w1_ref, b1_ref,
             w2_ref, b2_ref, w3_ref, b3_ref, o_ref):
    deg = deg_ref[:, 0:1] + deg_ref[:, 1:2] + 1.0
    dis = lax.rsqrt(deg)
    h = h_ref[...]
    out = dis * (acc_ref[0] + acc_ref[1]) + (dis * dis) * h + bc_ref[...]
    out = _leaky(out) + x_ref[...]
    h1 = _leaky(jnp.dot(out, w1_ref[...],
                        preferred_element_type=jnp.float32) + b1_ref[...])
    h2 = _leaky(jnp.dot(h1, w2_ref[...],
                        preferred_element_type=jnp.float32) + b2_ref[...])
    o_ref[...] = jnp.dot(h2, w3_ref[...],
                         preferred_element_type=jnp.float32) + b3_ref[...]


@jax.jit
def kernel(x, edge_index, edge_attr, Wc, bc, W1, b1, W2, b2, W3, b3):
    src = edge_index[0].astype(jnp.int32).reshape(NW, EPW)
    dst = edge_index[1].astype(jnp.int32).reshape(NW, EPW)
    wv = edge_attr.astype(jnp.float32).reshape(NW, EPW)

    deg = _deg_kernel(dst, wv)

    h = pl.pallas_call(
        _mm_body,
        grid=(N // _RB,),
        in_specs=[pl.BlockSpec((_RB, D), lambda i: (i, 0)),
                  pl.BlockSpec((D, D), lambda i: (0, 0))],
        out_specs=pl.BlockSpec((_RB, D), lambda i: (i, 0)),
        out_shape=jax.ShapeDtypeStruct((N, D), jnp.float32),
    )(x, Wc)

    acc = _msg_kernel(src, dst, wv, h, deg)

    degt = deg.reshape(NC, DEGR * 16)[:, :N].T

    w1p = jnp.zeros((D, D), jnp.float32).at[:, :8].set(W1)
    b1p = jnp.zeros((1, D), jnp.float32).at[0, :8].set(b1)
    w2p = jnp.zeros((D, D), jnp.float32).at[:8, :8].set(W2)
    b2p = jnp.zeros((1, D), jnp.float32).at[0, :8].set(b2)
    w3p = jnp.zeros((D, D), jnp.float32).at[:8, :1].set(W3)
    b3p = jnp.zeros((1, D), jnp.float32).at[0, :1].set(b3)

    full = pl.pallas_call(
        _ep_body,
        grid=(N // _RB,),
        in_specs=[
            pl.BlockSpec((NC, _RB, D), lambda i: (0, i, 0)),
            pl.BlockSpec((_RB, D), lambda i: (i, 0)),
            pl.BlockSpec((_RB, NC), lambda i: (i, 0)),
            pl.BlockSpec((_RB, D), lambda i: (i, 0)),
            pl.BlockSpec((1, D), lambda i: (0, 0)),
            pl.BlockSpec((D, D), lambda i: (0, 0)),
            pl.BlockSpec((1, D), lambda i: (0, 0)),
            pl.BlockSpec((D, D), lambda i: (0, 0)),
            pl.BlockSpec((1, D), lambda i: (0, 0)),
            pl.BlockSpec((D, D), lambda i: (0, 0)),
            pl.BlockSpec((1, D), lambda i: (0, 0)),
        ],
        out_specs=pl.BlockSpec((_RB, D), lambda i: (i, 0)),
        out_shape=jax.ShapeDtypeStruct((N, D), jnp.float32),
    )(acc, h, degt, x, bc.reshape(1, D), w1p, b1p, w2p, b2p, w3p, b3p)

    return full[:, :1]


# speedup vs baseline: 21.1831x; 1.4369x over previous
import functools

import jax
import jax.numpy as jnp
from jax import lax
from jax.experimental import pallas as pl
from jax.experimental.pallas import tpu as pltpu
from jax.experimental.pallas import tpu_sc as plsc

N = 10000
E = 320000
D = 128
NC = 2
NS = 16
NW = NC * NS
EPW = E // NW
NB = EPW // 16
SB = 5
NSB = NB // SB
DEGR = 640
HALF = DEGR * 16 // 2
RPT = N // NS
ZR = 25
CHK = 2000
NCHK = EPW // CHK
SBC = CHK // (SB * 16)

_mesh = plsc.VectorSubcoreMesh(core_axis_name="c", subcore_axis_name="s")


def _rsqrt16(t):
    bits = lax.bitcast_convert_type(t, jnp.int32)
    y = lax.bitcast_convert_type(jnp.int32(0x5F3759DF) - (bits >> 1),
                                 jnp.float32)
    for _ in range(3):
        y = y * (1.5 - 0.5 * t * y * y)
    return y


@functools.partial(
    pl.kernel,
    out_type=jax.ShapeDtypeStruct((NC, DEGR, 16), jnp.float32),
    mesh=_mesh,
    compiler_params=pltpu.CompilerParams(needs_layout_passes=False, use_tc_tiling_on_sc=False),
    scratch_types=[
        pltpu.VMEM((EPW,), jnp.int32),
        pltpu.VMEM((EPW,), jnp.float32),
        pltpu.VMEM((16 * HALF + 16,), jnp.float32),
        pltpu.VMEM((DEGR, 16), jnp.float32),
        pltpu.VMEM((40, 16), jnp.float32),
        pltpu.VMEM((40, 16), jnp.float32),
        pltpu.VMEM_SHARED((NS, DEGR, 16), jnp.float32),
    ],
)
def _deg_kernel(dst_hbm, w_hbm, deg_out, dst_v, w_v, hist, deg_local,
                acc40, buf40, deg_sh):
    cid = lax.axis_index("c")
    sid = lax.axis_index("s")
    wid = sid * NC + cid

    pltpu.sync_copy(dst_hbm.at[wid], dst_v)
    pltpu.sync_copy(w_hbm.at[wid], w_v)

    zero = jnp.zeros((16,), jnp.float32)
    lane = lax.iota(jnp.int32, 16)

    def _zero_hist(i, _):
        hist[pl.ds(i * 16, 16)] = zero
        return 0
    lax.fori_loop(0, (16 * HALF + 16) // 16, _zero_hist, 0)

    for p in range(2):
        def _hist_blk(i, _):
            dvec = dst_v[pl.ds(i * 16, 16)]
            wvec = w_v[pl.ds(i * 16, 16)]
            rel = dvec - p * HALF
            mask = (rel >= 0) & (rel < HALF)
            idx = jnp.where(mask, lane * HALF + rel, 16 * HALF + lane)
            plsc.addupdate_scatter(hist, [idx], wvec)
            return 0
        lax.fori_loop(0, NB, _hist_blk, 0)

        def _red_row(r, _):
            acc = zero
            for j in range(16):
                acc = acc + hist[pl.ds(j * HALF + r * 16, 16)]
            deg_local[p * (HALF // 16) + r, :] = acc
            return 0
        lax.fori_loop(0, HALF // 16, _red_row, 0)
        if p == 0:
            lax.fori_loop(0, (16 * HALF + 16) // 16, _zero_hist, 0)

    pltpu.sync_copy(deg_local, deg_sh.at[sid])
    plsc.subcore_barrier()

    def _zero40(i, _):
        acc40[i, :] = zero
        return 0
    lax.fori_loop(0, 40, _zero40, 0)
    for t in range(NS):
        pltpu.sync_copy(deg_sh.at[t, pl.ds(sid * 40, 40)], buf40)

        def _add40(i, _):
            acc40[i, :] = acc40[i, :] + buf40[i, :]
            return 0
        lax.fori_loop(0, 40, _add40, 0)
    pltpu.sync_copy(acc40, deg_out.at[cid, pl.ds(sid * 40, 40)])


@functools.partial(
    pl.kernel,
    out_type=jax.ShapeDtypeStruct((NC, N, D), jnp.float32),
    mesh=_mesh,
    compiler_params=pltpu.CompilerParams(needs_layout_passes=False, use_tc_tiling_on_sc=False),
    scratch_types=[
        pltpu.VMEM((CHK,), jnp.int32),
        pltpu.VMEM((CHK,), jnp.int32),
        pltpu.VMEM((CHK,), jnp.float32),
        pltpu.VMEM((8, 16), jnp.float32),
        pltpu.VMEM((DEGR, 16), jnp.float32),
        pltpu.VMEM((3, SB * 16), jnp.int32),
        pltpu.VMEM((3, SB * 16), jnp.int32),
        pltpu.VMEM((3, SB * 16), jnp.float32),
        pltpu.VMEM((3, SB * 16, D), jnp.float32),
        pltpu.VMEM((ZR, D), jnp.float32),
        pltpu.VMEM_SHARED((N, D), jnp.float32),
        pltpu.SemaphoreType.DMA,
        pltpu.SemaphoreType.DMA((3,)),
        pltpu.SemaphoreType.DMA((3,)),
    ],
)
def _msg_kernel(src_hbm, dst_hbm, w_hbm, h_hbm, deg_hbm, acc_out,
                src_c, dst_c, w_c, degbuf, dis_v, sidx, didx, fbuf, rows,
                zbuf, acc_sh, esem, gsem, wsem):
    cid = lax.axis_index("c")
    sid = lax.axis_index("s")
    wid = sid * NC + cid

    zero = jnp.zeros((16,), jnp.float32)

    pltpu.sync_copy(deg_hbm.at[0], dis_v)

    def _dis_chunk(b, _):
        pltpu.sync_copy(deg_hbm.at[1, pl.ds(b * 8, 8)], degbuf)

        def _dis_row(i, _):
            t = dis_v[b * 8 + i, :] + degbuf[i, :] + 1.0
            dis_v[b * 8 + i, :] = _rsqrt16(t)
            return 0
        lax.fori_loop(0, 8, _dis_row, 0)
        return 0
    lax.fori_loop(0, DEGR // 8, _dis_chunk, 0)

    def _zero_zbuf(i, _):
        for f in range(D // 16):
            zbuf[i, pl.ds(f * 16, 16)] = zero
        return 0
    lax.fori_loop(0, ZR, _zero_zbuf, 0)

    def _zero_acc(k, _):
        pltpu.sync_copy(zbuf, acc_sh.at[pl.ds(sid * RPT + k * ZR, ZR)])
        return 0
    lax.fori_loop(0, RPT // ZR, _zero_acc, 0)
    plsc.subcore_barrier()

    NSB_ALL = NSB

    def _stage(cidx):
        pltpu.async_copy(src_hbm.at[wid, pl.ds(cidx * CHK, CHK)], src_c, esem)
        pltpu.async_copy(dst_hbm.at[wid, pl.ds(cidx * CHK, CHK)], dst_c, esem)
        pltpu.async_copy(w_hbm.at[wid, pl.ds(cidx * CHK, CHK)], w_c, esem)
        pltpu.make_async_copy(src_hbm.at[wid, pl.ds(cidx * CHK, CHK)],
                              src_c, esem).wait()
        pltpu.make_async_copy(dst_hbm.at[wid, pl.ds(cidx * CHK, CHK)],
                              dst_c, esem).wait()
        pltpu.make_async_copy(w_hbm.at[wid, pl.ds(cidx * CHK, CHK)],
                              w_c, esem).wait()

    def _build(g, p):
        local = lax.rem(g, SBC)

        def _f(k, _):
            base = local * (SB * 16) + k * 16
            sv = src_c[pl.ds(base, 16)]
            sidx[p, pl.ds(k * 16, 16)] = sv
            didx[p, pl.ds(k * 16, 16)] = dst_c[pl.ds(base, 16)]
            fbuf[p, pl.ds(k * 16, 16)] = w_c[pl.ds(base, 16)] * (
                plsc.load_gather(dis_v, [sv >> 4, sv & 15]))
            return 0
        lax.fori_loop(0, SB, _f, 0)

    def _scale_scatter(q):
        def _scale(j, _):
            fj = plsc.load_gather(fbuf.at[q], [jnp.full((16,), j, jnp.int32)])
            for f in range(D // 16):
                rows[q, j, pl.ds(f * 16, 16)] = (
                    rows[q, j, pl.ds(f * 16, 16)] * fj)
            return 0
        lax.fori_loop(0, SB * 16, _scale, 0, unroll=2)
        pltpu.async_copy(rows.at[q], acc_sh.at[didx.at[q]], wsem.at[q],
                         add=True)

    def _step(g, _):
        p = lax.rem(g, 3)

        @pl.when(lax.rem(g, SBC) == 0)
        def _():
            _stage(lax.div(g, SBC))

        @pl.when(g >= 3)
        def _():
            pltpu.make_async_copy(rows.at[p], acc_sh.at[didx.at[p]],
                                  wsem.at[p]).wait()

        _build(g, p)
        pltpu.async_copy(h_hbm.at[sidx.at[p]], rows.at[p], gsem.at[p])

        @pl.when(g >= 1)
        def _():
            q = lax.rem(g - 1, 3)
            pltpu.make_async_copy(h_hbm.at[sidx.at[q]], rows.at[q],
                                  gsem.at[q]).wait()
            _scale_scatter(q)
        return 0
    lax.fori_loop(0, NSB_ALL, _step, 0)

    qlast = lax.rem(NSB_ALL - 1, 3)
    pltpu.make_async_copy(h_hbm.at[sidx.at[qlast]], rows.at[qlast],
                          gsem.at[qlast]).wait()
    _scale_scatter(qlast)
    for q in range(3):
        qd = lax.rem(NSB_ALL - 1 - (2 - q), 3)
        pltpu.make_async_copy(rows.at[qd], acc_sh.at[didx.at[qd]],
                              wsem.at[qd]).wait()
    plsc.subcore_barrier()

    def _out(k, _):
        pltpu.sync_copy(acc_sh.at[pl.ds(sid * RPT + k * ZR, ZR)], zbuf)
        pltpu.sync_copy(zbuf, acc_out.at[cid, pl.ds(sid * RPT + k * ZR, ZR)])
        return 0
    lax.fori_loop(0, RPT // ZR, _out, 0)


_RB = 2000


def _mm_body(x_ref, w_ref, o_ref):
    o_ref[...] = jnp.dot(x_ref[...], w_ref[...],
                         preferred_element_type=jnp.float32)


def _leaky(v):
    return jnp.where(v >= 0, v, 0.01 * v)


def _ep_body(acc_ref, h_ref, deg_ref, x_ref, bc_ref, w1_ref, b1_ref,
             w2_ref, b2_ref, w3_ref, b3_ref, o_ref):
    deg = deg_ref[:, 0:1] + deg_ref[:, 1:2] + 1.0
    dis = lax.rsqrt(deg)
    h = h_ref[...]
    out = dis * (acc_ref[0] + acc_ref[1]) + (dis * dis) * h + bc_ref[...]
    out = _leaky(out) + x_ref[...]
    h1 = _leaky(jnp.dot(out, w1_ref[...],
                        preferred_element_type=jnp.float32) + b1_ref[...])
    h2 = _leaky(jnp.dot(h1, w2_ref[...],
                        preferred_element_type=jnp.float32) + b2_ref[...])
    o_ref[...] = jnp.dot(h2, w3_ref[...],
                         preferred_element_type=jnp.float32) + b3_ref[...]


@jax.jit
def kernel(x, edge_index, edge_attr, Wc, bc, W1, b1, W2, b2, W3, b3):
    src = edge_index[0].astype(jnp.int32).reshape(NW, EPW)
    dst = edge_index[1].astype(jnp.int32).reshape(NW, EPW)
    wv = edge_attr.astype(jnp.float32).reshape(NW, EPW)

    deg = _deg_kernel(dst, wv)

    h = pl.pallas_call(
        _mm_body,
        grid=(N // _RB,),
        in_specs=[pl.BlockSpec((_RB, D), lambda i: (i, 0)),
                  pl.BlockSpec((D, D), lambda i: (0, 0))],
        out_specs=pl.BlockSpec((_RB, D), lambda i: (i, 0)),
        out_shape=jax.ShapeDtypeStruct((N, D), jnp.float32),
    )(x, Wc)

    acc = _msg_kernel(src, dst, wv, h, deg)

    degt = deg.reshape(NC, DEGR * 16)[:, :N].T

    w1p = jnp.zeros((D, D), jnp.float32).at[:, :8].set(W1)
    b1p = jnp.zeros((1, D), jnp.float32).at[0, :8].set(b1)
    w2p = jnp.zeros((D, D), jnp.float32).at[:8, :8].set(W2)
    b2p = jnp.zeros((1, D), jnp.float32).at[0, :8].set(b2)
    w3p = jnp.zeros((D, D), jnp.float32).at[:8, :1].set(W3)
    b3p = jnp.zeros((1, D), jnp.float32).at[0, :1].set(b3)

    full = pl.pallas_call(
        _ep_body,
        grid=(N // _RB,),
        in_specs=[
            pl.BlockSpec((NC, _RB, D), lambda i: (0, i, 0)),
            pl.BlockSpec((_RB, D), lambda i: (i, 0)),
            pl.BlockSpec((_RB, NC), lambda i: (i, 0)),
            pl.BlockSpec((_RB, D), lambda i: (i, 0)),
            pl.BlockSpec((1, D), lambda i: (0, 0)),
            pl.BlockSpec((D, D), lambda i: (0, 0)),
            pl.BlockSpec((1, D), lambda i: (0, 0)),
            pl.BlockSpec((D, D), lambda i: (0, 0)),
            pl.BlockSpec((1, D), lambda i: (0, 0)),
            pl.BlockSpec((D, D), lambda i: (0, 0)),
            pl.BlockSpec((1, D), lambda i: (0, 0)),
        ],
        out_specs=pl.BlockSpec((_RB, D), lambda i: (i, 0)),
        out_shape=jax.ShapeDtypeStruct((N, D), jnp.float32),
    )(acc, h, degt, x, bc.reshape(1, D), w1p, b1p, w2p, b2p, w3p, b3p)

    return full[:, :1]


# speedup vs baseline: 21.7191x; 1.0253x over previous
import functools

import jax
import jax.numpy as jnp
from jax import lax
from jax.experimental import pallas as pl
from jax.experimental.pallas import tpu as pltpu
from jax.experimental.pallas import tpu_sc as plsc

N = 10000
E = 320000
D = 128
NC = 2
NS = 16
NW = NC * NS
EPW = E // NW
NB = EPW // 16
SB = 5
NSB = NB // SB
DEGR = 640
HALF = DEGR * 16 // 2
RPT = N // NS
ZR = 25
CHK = 2000
NCHK = EPW // CHK
SBC = CHK // (SB * 16)

_mesh = plsc.VectorSubcoreMesh(core_axis_name="c", subcore_axis_name="s")


def _rsqrt16(t):
    bits = lax.bitcast_convert_type(t, jnp.int32)
    y = lax.bitcast_convert_type(jnp.int32(0x5F3759DF) - (bits >> 1),
                                 jnp.float32)
    for _ in range(3):
        y = y * (1.5 - 0.5 * t * y * y)
    return y


@functools.partial(
    pl.kernel,
    out_type=jax.ShapeDtypeStruct((NC, DEGR, 16), jnp.float32),
    mesh=_mesh,
    compiler_params=pltpu.CompilerParams(needs_layout_passes=False, use_tc_tiling_on_sc=False),
    scratch_types=[
        pltpu.VMEM((EPW,), jnp.int32),
        pltpu.VMEM((EPW,), jnp.float32),
        pltpu.VMEM((16 * HALF + 16,), jnp.float32),
        pltpu.VMEM((DEGR, 16), jnp.float32),
        pltpu.VMEM((DEGR // 128, 128), jnp.int32),
        pltpu.VMEM((40, 16), jnp.float32),
        pltpu.VMEM_SHARED((DEGR, 16), jnp.float32),
        pltpu.SemaphoreType.DMA,
    ],
)
def _deg_kernel(dst_hbm, w_hbm, deg_out, dst_v, w_v, hist, deg_local,
                iota_v, zbuf40, deg_sh, sem):
    cid = lax.axis_index("c")
    sid = lax.axis_index("s")
    wid = sid * NC + cid

    pltpu.sync_copy(dst_hbm.at[wid], dst_v)
    pltpu.sync_copy(w_hbm.at[wid], w_v)

    zero = jnp.zeros((16,), jnp.float32)
    lane = lax.iota(jnp.int32, 16)

    def _zero_hist(i, _):
        hist[pl.ds(i * 16, 16)] = zero
        return 0
    lax.fori_loop(0, (16 * HALF + 16) // 16, _zero_hist, 0)

    for p in range(2):
        def _hist_blk(i, _):
            dvec = dst_v[pl.ds(i * 16, 16)]
            wvec = w_v[pl.ds(i * 16, 16)]
            rel = dvec - p * HALF
            mask = (rel >= 0) & (rel < HALF)
            idx = jnp.where(mask, lane * HALF + rel, 16 * HALF + lane)
            plsc.addupdate_scatter(hist, [idx], wvec)
            return 0
        lax.fori_loop(0, NB, _hist_blk, 0)

        def _red_row(r, _):
            acc = zero
            for j in range(16):
                acc = acc + hist[pl.ds(j * HALF + r * 16, 16)]
            deg_local[p * (HALF // 16) + r, :] = acc
            return 0
        lax.fori_loop(0, HALF // 16, _red_row, 0)
        if p == 0:
            lax.fori_loop(0, (16 * HALF + 16) // 16, _zero_hist, 0)

    for k in range(DEGR // 128):
        for f in range(8):
            iota_v[k, pl.ds(f * 16, 16)] = (
                lane + k * 128 + f * 16)

    def _zero40(i, _):
        zbuf40[i, :] = zero
        return 0
    lax.fori_loop(0, 40, _zero40, 0)
    pltpu.sync_copy(zbuf40, deg_sh.at[pl.ds(sid * 40, 40)])
    plsc.subcore_barrier()
    for k in range(DEGR // 128):
        pltpu.async_copy(deg_local.at[pl.ds(k * 128, 128)],
                         deg_sh.at[iota_v.at[k]], sem, add=True)
    for k in range(DEGR // 128):
        pltpu.make_async_copy(deg_local.at[pl.ds(k * 128, 128)],
                              deg_sh.at[iota_v.at[k]], sem).wait()
    plsc.subcore_barrier()
    pltpu.sync_copy(deg_sh.at[pl.ds(sid * 40, 40)],
                    deg_out.at[cid, pl.ds(sid * 40, 40)])


@functools.partial(
    pl.kernel,
    out_type=jax.ShapeDtypeStruct((NC, N, D), jnp.float32),
    mesh=_mesh,
    compiler_params=pltpu.CompilerParams(needs_layout_passes=False, use_tc_tiling_on_sc=False),
    scratch_types=[
        pltpu.VMEM((CHK,), jnp.int32),
        pltpu.VMEM((CHK,), jnp.int32),
        pltpu.VMEM((CHK,), jnp.float32),
        pltpu.VMEM((8, 16), jnp.float32),
        pltpu.VMEM((DEGR, 16), jnp.float32),
        pltpu.VMEM((3, SB * 16), jnp.int32),
        pltpu.VMEM((3, SB * 16), jnp.int32),
        pltpu.VMEM((3, SB * 16), jnp.float32),
        pltpu.VMEM((3, SB * 16, D), jnp.float32),
        pltpu.VMEM((ZR, D), jnp.float32),
        pltpu.VMEM_SHARED((N, D), jnp.float32),
        pltpu.SemaphoreType.DMA,
        pltpu.SemaphoreType.DMA((3,)),
        pltpu.SemaphoreType.DMA((3,)),
    ],
)
def _msg_kernel(src_hbm, dst_hbm, w_hbm, h_hbm, deg_hbm, acc_out,
                src_c, dst_c, w_c, degbuf, dis_v, sidx, didx, fbuf, rows,
                zbuf, acc_sh, esem, gsem, wsem):
    cid = lax.axis_index("c")
    sid = lax.axis_index("s")
    wid = sid * NC + cid

    zero = jnp.zeros((16,), jnp.float32)

    pltpu.sync_copy(deg_hbm.at[0], dis_v)

    def _dis_chunk(b, _):
        pltpu.sync_copy(deg_hbm.at[1, pl.ds(b * 8, 8)], degbuf)

        def _dis_row(i, _):
            t = dis_v[b * 8 + i, :] + degbuf[i, :] + 1.0
            dis_v[b * 8 + i, :] = _rsqrt16(t)
            return 0
        lax.fori_loop(0, 8, _dis_row, 0)
        return 0
    lax.fori_loop(0, DEGR // 8, _dis_chunk, 0)

    def _zero_zbuf(i, _):
        for f in range(D // 16):
            zbuf[i, pl.ds(f * 16, 16)] = zero
        return 0
    lax.fori_loop(0, ZR, _zero_zbuf, 0)

    def _zero_acc(k, _):
        pltpu.async_copy(zbuf, acc_sh.at[pl.ds(sid * RPT + k * ZR, ZR)], esem)
        return 0
    lax.fori_loop(0, RPT // ZR, _zero_acc, 0)

    def _zero_drain(k, _):
        pltpu.make_async_copy(
            zbuf, acc_sh.at[pl.ds(sid * RPT + k * ZR, ZR)], esem).wait()
        return 0
    lax.fori_loop(0, RPT // ZR, _zero_drain, 0)
    plsc.subcore_barrier()

    NSB_ALL = NSB

    def _stage(cidx):
        pltpu.async_copy(src_hbm.at[wid, pl.ds(cidx * CHK, CHK)], src_c, esem)
        pltpu.async_copy(dst_hbm.at[wid, pl.ds(cidx * CHK, CHK)], dst_c, esem)
        pltpu.async_copy(w_hbm.at[wid, pl.ds(cidx * CHK, CHK)], w_c, esem)
        pltpu.make_async_copy(src_hbm.at[wid, pl.ds(cidx * CHK, CHK)],
                              src_c, esem).wait()
        pltpu.make_async_copy(dst_hbm.at[wid, pl.ds(cidx * CHK, CHK)],
                              dst_c, esem).wait()
        pltpu.make_async_copy(w_hbm.at[wid, pl.ds(cidx * CHK, CHK)],
                              w_c, esem).wait()

    def _build(g, p):
        local = lax.rem(g, SBC)

        def _f(k, _):
            base = local * (SB * 16) + k * 16
            sv = src_c[pl.ds(base, 16)]
            sidx[p, pl.ds(k * 16, 16)] = sv
            didx[p, pl.ds(k * 16, 16)] = dst_c[pl.ds(base, 16)]
            fbuf[p, pl.ds(k * 16, 16)] = w_c[pl.ds(base, 16)] * (
                plsc.load_gather(dis_v, [sv >> 4, sv & 15]))
            return 0
        lax.fori_loop(0, SB, _f, 0)

    def _scale_scatter(q):
        def _scale(j, _):
            fj = plsc.load_gather(fbuf.at[q], [jnp.full((16,), j, jnp.int32)])
            for f in range(D // 16):
                rows[q, j, pl.ds(f * 16, 16)] = (
                    rows[q, j, pl.ds(f * 16, 16)] * fj)
            return 0
        lax.fori_loop(0, SB * 16, _scale, 0, unroll=2)
        pltpu.async_copy(rows.at[q], acc_sh.at[didx.at[q]], wsem.at[q],
                         add=True)

    def _step(g, _):
        p = lax.rem(g, 3)

        @pl.when(lax.rem(g, SBC) == 0)
        def _():
            _stage(lax.div(g, SBC))

        @pl.when(g >= 3)
        def _():
            pltpu.make_async_copy(rows.at[p], acc_sh.at[didx.at[p]],
                                  wsem.at[p]).wait()

        _build(g, p)
        pltpu.async_copy(h_hbm.at[sidx.at[p]], rows.at[p], gsem.at[p])

        @pl.when(g >= 1)
        def _():
            q = lax.rem(g - 1, 3)
            pltpu.make_async_copy(h_hbm.at[sidx.at[q]], rows.at[q],
                                  gsem.at[q]).wait()
            _scale_scatter(q)
        return 0
    lax.fori_loop(0, NSB_ALL, _step, 0)

    qlast = lax.rem(NSB_ALL - 1, 3)
    pltpu.make_async_copy(h_hbm.at[sidx.at[qlast]], rows.at[qlast],
                          gsem.at[qlast]).wait()
    _scale_scatter(qlast)
    for q in range(3):
        qd = lax.rem(NSB_ALL - 1 - (2 - q), 3)
        pltpu.make_async_copy(rows.at[qd], acc_sh.at[didx.at[qd]],
                              wsem.at[qd]).wait()
    plsc.subcore_barrier()

    pltpu.sync_copy(acc_sh.at[pl.ds(sid * RPT, RPT)],
                    acc_out.at[cid, pl.ds(sid * RPT, RPT)])


_RB = 2000


def _mm_body(x_ref, w_ref, o_ref):
    o_ref[...] = jnp.dot(x_ref[...], w_ref[...],
                         preferred_element_type=jnp.float32)


def _leaky(v):
    return jnp.where(v >= 0, v, 0.01 * v)


def _ep_body(acc_ref, h_ref, deg_ref, x_ref, bc_ref, w1_ref, b1_ref,
             w2_ref, b2_ref, w3_ref, b3_ref, o_ref):
    deg = deg_ref[:, 0:1] + deg_ref[:, 1:2] + 1.0
    dis = lax.rsqrt(deg)
    h = h_ref[...]
    out = dis * (acc_ref[0] + acc_ref[1]) + (dis * dis) * h + bc_ref[...]
    out = _leaky(out) + x_ref[...]
    h1 = _leaky(jnp.dot(out, w1_ref[...],
                        preferred_element_type=jnp.float32) + b1_ref[...])
    h2 = _leaky(jnp.dot(h1, w2_ref[...],
                        preferred_element_type=jnp.float32) + b2_ref[...])
    o_ref[...] = jnp.dot(h2, w3_ref[...],
                         preferred_element_type=jnp.float32) + b3_ref[...]


@jax.jit
def kernel(x, edge_index, edge_attr, Wc, bc, W1, b1, W2, b2, W3, b3):
    src = edge_index[0].astype(jnp.int32).reshape(NW, EPW)
    dst = edge_index[1].astype(jnp.int32).reshape(NW, EPW)
    wv = edge_attr.astype(jnp.float32).reshape(NW, EPW)

    deg = _deg_kernel(dst, wv)

    h = pl.pallas_call(
        _mm_body,
        grid=(N // _RB,),
        in_specs=[pl.BlockSpec((_RB, D), lambda i: (i, 0)),
                  pl.BlockSpec((D, D), lambda i: (0, 0))],
        out_specs=pl.BlockSpec((_RB, D), lambda i: (i, 0)),
        out_shape=jax.ShapeDtypeStruct((N, D), jnp.float32),
    )(x, Wc)

    acc = _msg_kernel(src, dst, wv, h, deg)

    degt = deg.reshape(NC, DEGR * 16)[:, :N].T

    w1p = jnp.zeros((D, D), jnp.float32).at[:, :8].set(W1)
    b1p = jnp.zeros((1, D), jnp.float32).at[0, :8].set(b1)
    w2p = jnp.zeros((D, D), jnp.float32).at[:8, :8].set(W2)
    b2p = jnp.zeros((1, D), jnp.float32).at[0, :8].set(b2)
    w3p = jnp.zeros((D, D), jnp.float32).at[:8, :1].set(W3)
    b3p = jnp.zeros((1, D), jnp.float32).at[0, :1].set(b3)

    full = pl.pallas_call(
        _ep_body,
        grid=(N // _RB,),
        in_specs=[
            pl.BlockSpec((NC, _RB, D), lambda i: (0, i, 0)),
            pl.BlockSpec((_RB, D), lambda i: (i, 0)),
            pl.BlockSpec((_RB, NC), lambda i: (i, 0)),
            pl.BlockSpec((_RB, D), lambda i: (i, 0)),
            pl.BlockSpec((1, D), lambda i: (0, 0)),
            pl.BlockSpec((D, D), lambda i: (0, 0)),
            pl.BlockSpec((1, D), lambda i: (0, 0)),
            pl.BlockSpec((D, D), lambda i: (0, 0)),
            pl.BlockSpec((1, D), lambda i: (0, 0)),
            pl.BlockSpec((D, D), lambda i: (0, 0)),
            pl.BlockSpec((1, D), lambda i: (0, 0)),
        ],
        out_specs=pl.BlockSpec((_RB, D), lambda i: (i, 0)),
        out_shape=jax.ShapeDtypeStruct((N, D), jnp.float32),
    )(acc, h, degt, x, bc.reshape(1, D), w1p, b1p, w2p, b2p, w3p, b3p)

    return full[:, :1]


# speedup vs baseline: 30.1373x; 1.3876x over previous
import functools

import jax
import jax.numpy as jnp
from jax import lax
from jax.experimental import pallas as pl
from jax.experimental.pallas import tpu as pltpu
from jax.experimental.pallas import tpu_sc as plsc

N = 10000
E = 320000
D = 128
NC = 2
NS = 16
NW = NC * NS
EPW = E // NW
NB = EPW // 16
SB = 5
NSB = NB // SB
DEGR = 640
HALF = DEGR * 16 // 2
RPT = N // NS
ZR = 25
CHK = 2000
NCHK = EPW // CHK
SBC = CHK // (SB * 16)

_mesh = plsc.VectorSubcoreMesh(core_axis_name="c", subcore_axis_name="s")


def _rsqrt16(t):
    bits = lax.bitcast_convert_type(t, jnp.int32)
    y = lax.bitcast_convert_type(jnp.int32(0x5F3759DF) - (bits >> 1),
                                 jnp.float32)
    for _ in range(3):
        y = y * (1.5 - 0.5 * t * y * y)
    return y


@functools.partial(
    pl.kernel,
    out_type=jax.ShapeDtypeStruct((NC, DEGR, 16), jnp.float32),
    mesh=_mesh,
    compiler_params=pltpu.CompilerParams(needs_layout_passes=False, use_tc_tiling_on_sc=False),
    scratch_types=[
        pltpu.VMEM((EPW,), jnp.int32),
        pltpu.VMEM((EPW,), jnp.float32),
        pltpu.VMEM((16 * HALF + 16,), jnp.float32),
        pltpu.VMEM((DEGR, 16), jnp.float32),
        pltpu.VMEM((DEGR // 128, 128), jnp.int32),
        pltpu.VMEM((40, 16), jnp.float32),
        pltpu.VMEM_SHARED((DEGR, 16), jnp.float32),
        pltpu.SemaphoreType.DMA,
    ],
)
def _deg_kernel(dst_hbm, w_hbm, deg_out, dst_v, w_v, hist, deg_local,
                iota_v, zbuf40, deg_sh, sem):
    cid = lax.axis_index("c")
    sid = lax.axis_index("s")
    wid = sid * NC + cid

    pltpu.sync_copy(dst_hbm.at[wid], dst_v)
    pltpu.sync_copy(w_hbm.at[wid], w_v)

    zero = jnp.zeros((16,), jnp.float32)
    lane = lax.iota(jnp.int32, 16)

    def _zero_hist(i, _):
        hist[pl.ds(i * 16, 16)] = zero
        return 0
    lax.fori_loop(0, (16 * HALF + 16) // 16, _zero_hist, 0, unroll=8)

    for p in range(2):
        def _hist_blk(i, _):
            dvec = dst_v[pl.ds(i * 16, 16)]
            wvec = w_v[pl.ds(i * 16, 16)]
            rel = dvec - p * HALF
            mask = (rel >= 0) & (rel < HALF)
            idx = jnp.where(mask, lane * HALF + rel, 16 * HALF + lane)
            plsc.addupdate_scatter(hist, [idx], wvec)
            return 0
        lax.fori_loop(0, NB, _hist_blk, 0, unroll=2)

        def _red_row(r, _):
            acc = zero
            for j in range(16):
                acc = acc + hist[pl.ds(j * HALF + r * 16, 16)]
            deg_local[p * (HALF // 16) + r, :] = acc
            return 0
        lax.fori_loop(0, HALF // 16, _red_row, 0, unroll=2)
        if p == 0:
            lax.fori_loop(0, (16 * HALF + 16) // 16, _zero_hist, 0, unroll=8)

    for k in range(DEGR // 128):
        for f in range(8):
            iota_v[k, pl.ds(f * 16, 16)] = (
                lane + k * 128 + f * 16)

    def _zero40(i, _):
        zbuf40[i, :] = zero
        return 0
    lax.fori_loop(0, 40, _zero40, 0)
    pltpu.sync_copy(zbuf40, deg_sh.at[pl.ds(sid * 40, 40)])
    plsc.subcore_barrier()
    for k in range(DEGR // 128):
        pltpu.async_copy(deg_local.at[pl.ds(k * 128, 128)],
                         deg_sh.at[iota_v.at[k]], sem, add=True)
    for k in range(DEGR // 128):
        pltpu.make_async_copy(deg_local.at[pl.ds(k * 128, 128)],
                              deg_sh.at[iota_v.at[k]], sem).wait()
    plsc.subcore_barrier()
    pltpu.sync_copy(deg_sh.at[pl.ds(sid * 40, 40)],
                    deg_out.at[cid, pl.ds(sid * 40, 40)])


@functools.partial(
    pl.kernel,
    out_type=jax.ShapeDtypeStruct((NC, N, D), jnp.float32),
    mesh=_mesh,
    compiler_params=pltpu.CompilerParams(needs_layout_passes=False,
                                         use_tc_tiling_on_sc=False),
    scratch_types=[
        pltpu.VMEM((2, CHK), jnp.int32),
        pltpu.VMEM((2, CHK), jnp.int32),
        pltpu.VMEM((2, CHK), jnp.float32),
        pltpu.VMEM((3, SB * 16), jnp.int32),
        pltpu.VMEM((3, SB * 16), jnp.int32),
        pltpu.VMEM((3, SB * 16, D), jnp.float32),
        pltpu.VMEM((ZR, D), jnp.float32),
        pltpu.VMEM_SHARED((N, D), jnp.float32),
        pltpu.SemaphoreType.DMA,
        pltpu.SemaphoreType.DMA,
        pltpu.SemaphoreType.DMA((3,)),
        pltpu.SemaphoreType.DMA((3,)),
    ],
)
def _msg_kernel(src_hbm, dst_hbm, w_hbm, h_hbm, acc_out,
                src_c, dst_c, w_c, sidx, didx, rows,
                zbuf, acc_sh, esem, psem, gsem, wsem):
    cid = lax.axis_index("c")
    sid = lax.axis_index("s")
    wid = sid * NC + cid

    zero = jnp.zeros((16,), jnp.float32)

    def _zero_zbuf(i, _):
        for f in range(D // 16):
            zbuf[i, pl.ds(f * 16, 16)] = zero
        return 0
    lax.fori_loop(0, ZR, _zero_zbuf, 0, unroll=4)

    def _zero_acc(k, _):
        pltpu.async_copy(zbuf, acc_sh.at[pl.ds(sid * RPT + k * ZR, ZR)], esem)
        return 0
    lax.fori_loop(0, RPT // ZR, _zero_acc, 0)

    def _stage(cidx, slot):
        pltpu.async_copy(src_hbm.at[wid, pl.ds(cidx * CHK, CHK)],
                         src_c.at[slot], psem)
        pltpu.async_copy(dst_hbm.at[wid, pl.ds(cidx * CHK, CHK)],
                         dst_c.at[slot], psem)
        pltpu.async_copy(w_hbm.at[wid, pl.ds(cidx * CHK, CHK)],
                         w_c.at[slot], psem)

    def _stage_wait(cidx, slot):
        pltpu.make_async_copy(src_hbm.at[wid, pl.ds(cidx * CHK, CHK)],
                              src_c.at[slot], psem).wait()
        pltpu.make_async_copy(dst_hbm.at[wid, pl.ds(cidx * CHK, CHK)],
                              dst_c.at[slot], psem).wait()
        pltpu.make_async_copy(w_hbm.at[wid, pl.ds(cidx * CHK, CHK)],
                              w_c.at[slot], psem).wait()

    _stage(0, 0)
    _stage_wait(0, 0)

    def _zero_drain(k, _):
        pltpu.make_async_copy(
            zbuf, acc_sh.at[pl.ds(sid * RPT + k * ZR, ZR)], esem).wait()
        return 0
    lax.fori_loop(0, RPT // ZR, _zero_drain, 0)
    plsc.subcore_barrier()

    def _build(lc, slot, p):
        def _f(k, _):
            base = lc * (SB * 16) + k * 16
            sidx[p, pl.ds(k * 16, 16)] = src_c[slot, pl.ds(base, 16)]
            didx[p, pl.ds(k * 16, 16)] = dst_c[slot, pl.ds(base, 16)]
            return 0
        lax.fori_loop(0, SB, _f, 0)

    def _scale_scatter(gq, q):
        lcq = lax.rem(gq, SBC)
        slotq = lax.rem(lax.div(gq, SBC), 2)

        def _scale(j, _):
            fj = plsc.load_gather(
                w_c, [jnp.full((16,), slotq, jnp.int32),
                      jnp.full((16,), lcq * (SB * 16) + j, jnp.int32)])
            for f in range(D // 16):
                rows[q, j, pl.ds(f * 16, 16)] = (
                    rows[q, j, pl.ds(f * 16, 16)] * fj)
            return 0
        lax.fori_loop(0, SB * 16, _scale, 0, unroll=4)
        pltpu.async_copy(rows.at[q], acc_sh.at[didx.at[q]], wsem.at[q],
                         add=True)

    def _step(g, _):
        p = lax.rem(g, 3)
        c = lax.div(g, SBC)
        lc = lax.rem(g, SBC)
        slot = lax.rem(c, 2)

        @pl.when(jnp.logical_and(lc == 1, c < NCHK - 1))
        def _():
            _stage(c + 1, lax.rem(c + 1, 2))

        @pl.when(jnp.logical_and(lc == 0, g > 0))
        def _():
            _stage_wait(c, slot)

        @pl.when(g >= 3)
        def _():
            pltpu.make_async_copy(rows.at[p], acc_sh.at[didx.at[p]],
                                  wsem.at[p]).wait()

        _build(lc, slot, p)
        pltpu.async_copy(h_hbm.at[sidx.at[p]], rows.at[p], gsem.at[p])

        @pl.when(g >= 1)
        def _():
            q = lax.rem(g - 1, 3)
            pltpu.make_async_copy(h_hbm.at[sidx.at[q]], rows.at[q],
                                  gsem.at[q]).wait()
            _scale_scatter(g - 1, q)
        return 0
    lax.fori_loop(0, NSB, _step, 0)

    qlast = lax.rem(NSB - 1, 3)
    pltpu.make_async_copy(h_hbm.at[sidx.at[qlast]], rows.at[qlast],
                          gsem.at[qlast]).wait()
    _scale_scatter(NSB - 1, qlast)
    for q in range(3):
        qd = lax.rem(NSB - 1 - (2 - q), 3)
        pltpu.make_async_copy(rows.at[qd], acc_sh.at[didx.at[qd]],
                              wsem.at[qd]).wait()
    plsc.subcore_barrier()

    pltpu.sync_copy(acc_sh.at[pl.ds(sid * RPT, RPT)],
                    acc_out.at[cid, pl.ds(sid * RPT, RPT)])


_RB = 2000


def _mm_body(x_ref, w_ref, deg_ref, o_ref):
    deg = deg_ref[:, 0:1] + deg_ref[:, 1:2] + 1.0
    dis = lax.rsqrt(deg)
    o_ref[...] = dis * jnp.dot(x_ref[...], w_ref[...],
                               preferred_element_type=jnp.float32)


def _leaky(v):
    return jnp.where(v >= 0, v, 0.01 * v)


def _ep_body(acc_ref, h_ref, deg_ref, x_ref, bc_ref, w1_ref, b1_ref,
             w2_ref, b2_ref, w3_ref, b3_ref, o_ref):
    deg = deg_ref[:, 0:1] + deg_ref[:, 1:2] + 1.0
    dis = lax.rsqrt(deg)
    h = h_ref[...]
    out = dis * (acc_ref[0] + acc_ref[1] + h) + bc_ref[...]
    out = _leaky(out) + x_ref[...]
    h1 = _leaky(jnp.dot(out, w1_ref[...],
                        preferred_element_type=jnp.float32) + b1_ref[...])
    h2 = _leaky(jnp.dot(h1, w2_ref[...],
                        preferred_element_type=jnp.float32) + b2_ref[...])
    o_ref[...] = jnp.dot(h2, w3_ref[...],
                         preferred_element_type=jnp.float32) + b3_ref[...]


@jax.jit
def kernel(x, edge_index, edge_attr, Wc, bc, W1, b1, W2, b2, W3, b3):
    src = edge_index[0].astype(jnp.int32).reshape(NW, EPW)
    dst = edge_index[1].astype(jnp.int32).reshape(NW, EPW)
    wv = edge_attr.astype(jnp.float32).reshape(NW, EPW)

    deg = _deg_kernel(dst, wv)
    degt = deg.reshape(NC, DEGR * 16)[:, :N].T

    h = pl.pallas_call(
        _mm_body,
        grid=(N // _RB,),
        in_specs=[pl.BlockSpec((_RB, D), lambda i: (i, 0)),
                  pl.BlockSpec((D, D), lambda i: (0, 0)),
                  pl.BlockSpec((_RB, NC), lambda i: (i, 0))],
        out_specs=pl.BlockSpec((_RB, D), lambda i: (i, 0)),
        out_shape=jax.ShapeDtypeStruct((N, D), jnp.float32),
    )(x, Wc, degt)

    acc = _msg_kernel(src, dst, wv, h)

    w1p = jnp.zeros((D, D), jnp.float32).at[:, :8].set(W1)
    b1p = jnp.zeros((1, D), jnp.float32).at[0, :8].set(b1)
    w2p = jnp.zeros((D, D), jnp.float32).at[:8, :8].set(W2)
    b2p = jnp.zeros((1, D), jnp.float32).at[0, :8].set(b2)
    w3p = jnp.zeros((D, D), jnp.float32).at[:8, :1].set(W3)
    b3p = jnp.zeros((1, D), jnp.float32).at[0, :1].set(b3)

    full = pl.pallas_call(
        _ep_body,
        grid=(N // _RB,),
        in_specs=[
            pl.BlockSpec((NC, _RB, D), lambda i: (0, i, 0)),
            pl.BlockSpec((_RB, D), lambda i: (i, 0)),
            pl.BlockSpec((_RB, NC), lambda i: (i, 0)),
            pl.BlockSpec((_RB, D), lambda i: (i, 0)),
            pl.BlockSpec((1, D), lambda i: (0, 0)),
            pl.BlockSpec((D, D), lambda i: (0, 0)),
            pl.BlockSpec((1, D), lambda i: (0, 0)),
            pl.BlockSpec((D, D), lambda i: (0, 0)),
            pl.BlockSpec((1, D), lambda i: (0, 0)),
            pl.BlockSpec((D, D), lambda i: (0, 0)),
            pl.BlockSpec((1, D), lambda i: (0, 0)),
        ],
        out_specs=pl.BlockSpec((_RB, D), lambda i: (i, 0)),
        out_shape=jax.ShapeDtypeStruct((N, D), jnp.float32),
    )(acc, h, degt, x, bc.reshape(1, D), w1p, b1p, w2p, b2p, w3p, b3p)

    return full[:, :1]
